# baseline (speedup 1.0000x reference)
import jax
import jax.numpy as jnp
from jax.experimental import pallas as pl
from jax.experimental.pallas import tpu as pltpu

B = 4096
K = 64


def _body(in_hbm, xui_ref, gum_hbm, gim_hbm, s_v, sem_ld, sem_su, sem_si):
    ld = pltpu.make_async_copy(in_hbm, s_v, sem_ld)
    ld.start()
    ld.wait()
    st_u = pltpu.make_async_copy(s_v.at[0], gum_hbm, sem_su)
    st_i = pltpu.make_async_copy(s_v.at[1], gim_hbm, sem_si)
    st_u.start()
    st_i.start()
    xui_ref[...] = jnp.sum(s_v[0] * s_v[1], axis=1)
    st_u.wait()
    st_i.wait()


def kernel(inputs):
    xui, gum, gim = pl.pallas_call(
        _body,
        in_specs=[pl.BlockSpec(memory_space=pltpu.MemorySpace.HBM)],
        out_specs=[
            pl.BlockSpec(memory_space=pltpu.MemorySpace.VMEM),
            pl.BlockSpec(memory_space=pltpu.MemorySpace.HBM),
            pl.BlockSpec(memory_space=pltpu.MemorySpace.HBM),
        ],
        out_shape=[
            jax.ShapeDtypeStruct((B,), jnp.float32),
            jax.ShapeDtypeStruct((B, K), jnp.float32),
            jax.ShapeDtypeStruct((B, K), jnp.float32),
        ],
        scratch_shapes=[
            pltpu.VMEM((2, B, K), jnp.float32),
            pltpu.SemaphoreType.DMA,
            pltpu.SemaphoreType.DMA,
            pltpu.SemaphoreType.DMA,
        ],
    )(inputs)
    return (xui, gum, gim)


# speedup vs baseline: 1.0003x; 1.0003x over previous
import jax
import jax.numpy as jnp
from jax.experimental import pallas as pl
from jax.experimental.pallas import tpu as pltpu

B = 4096
K = 64


def _body(in_hbm, xui_ref, gum_hbm, gim_hbm, u_v, w_v,
          sem_lu, sem_lw, sem_su, sem_si):
    ld_u = pltpu.make_async_copy(in_hbm.at[0], u_v, sem_lu)
    ld_w = pltpu.make_async_copy(in_hbm.at[1], w_v, sem_lw)
    ld_u.start()
    ld_w.start()
    ld_u.wait()
    ld_w.wait()
    st_u = pltpu.make_async_copy(u_v, gum_hbm, sem_su)
    st_i = pltpu.make_async_copy(w_v, gim_hbm, sem_si)
    st_u.start()
    st_i.start()
    xui_ref[...] = jnp.sum(u_v[...] * w_v[...], axis=1)
    st_u.wait()
    st_i.wait()


def kernel(inputs):
    xui, gum, gim = pl.pallas_call(
        _body,
        in_specs=[pl.BlockSpec(memory_space=pltpu.MemorySpace.HBM)],
        out_specs=[
            pl.BlockSpec(memory_space=pltpu.MemorySpace.VMEM),
            pl.BlockSpec(memory_space=pltpu.MemorySpace.HBM),
            pl.BlockSpec(memory_space=pltpu.MemorySpace.HBM),
        ],
        out_shape=[
            jax.ShapeDtypeStruct((B,), jnp.float32),
            jax.ShapeDtypeStruct((B, K), jnp.float32),
            jax.ShapeDtypeStruct((B, K), jnp.float32),
        ],
        scratch_shapes=[
            pltpu.VMEM((B, K), jnp.float32),
            pltpu.VMEM((B, K), jnp.float32),
            pltpu.SemaphoreType.DMA,
            pltpu.SemaphoreType.DMA,
            pltpu.SemaphoreType.DMA,
            pltpu.SemaphoreType.DMA,
        ],
    )(inputs)
    return (xui, gum, gim)


# speedup vs baseline: 1.0795x; 1.0792x over previous
import jax
import jax.numpy as jnp
from jax.experimental import pallas as pl

B = 4096
K = 64
BLK = 512


def _body(in_ref, xui_ref):
    xui_ref[...] = jnp.sum(in_ref[0] * in_ref[1], axis=1)


def kernel(inputs):
    xui = pl.pallas_call(
        _body,
        grid=(B // BLK,),
        in_specs=[pl.BlockSpec((2, BLK, K), lambda i: (0, i, 0))],
        out_specs=pl.BlockSpec((BLK,), lambda i: (i,)),
        out_shape=jax.ShapeDtypeStruct((B,), jnp.float32),
    )(inputs)
    return (xui, inputs[0], inputs[1])


# speedup vs baseline: 1.3926x; 1.2900x over previous
import jax
import jax.numpy as jnp
from jax.experimental import pallas as pl
from jax.experimental.pallas import tpu as pltpu

B = 4096
K = 64
NCHUNK = 4
CHUNK = B // NCHUNK


def _body(in_hbm, xui_ref, u_v, w_v, *sems):
    copies = []
    for g in range(NCHUNK):
        rows = pl.ds(g * CHUNK, CHUNK)
        cu = pltpu.make_async_copy(in_hbm.at[0, rows], u_v.at[rows], sems[2 * g])
        cw = pltpu.make_async_copy(in_hbm.at[1, rows], w_v.at[rows], sems[2 * g + 1])
        cu.start()
        cw.start()
        copies.append((cu, cw))
    for g in range(NCHUNK):
        cu, cw = copies[g]
        cu.wait()
        cw.wait()
        rows = pl.ds(g * CHUNK, CHUNK)
        xui_ref[rows] = jnp.sum(u_v[rows, :] * w_v[rows, :], axis=1)


def kernel(inputs):
    xui = pl.pallas_call(
        _body,
        in_specs=[pl.BlockSpec(memory_space=pltpu.MemorySpace.HBM)],
        out_specs=pl.BlockSpec(memory_space=pltpu.MemorySpace.VMEM),
        out_shape=jax.ShapeDtypeStruct((B,), jnp.float32),
        scratch_shapes=[
            pltpu.VMEM((B, K), jnp.float32),
            pltpu.VMEM((B, K), jnp.float32),
        ] + [pltpu.SemaphoreType.DMA] * (2 * NCHUNK),
    )(inputs)
    return (xui, inputs[0], inputs[1])
